# baseline (speedup 1.0000x reference)
import functools

import jax
import jax.numpy as jnp
from jax import lax
from jax.experimental import pallas as pl
from jax.experimental.pallas import tpu as pltpu
from jax.experimental.pallas import tpu_sc as plsc

BATCH_SIZE = 16384
D0 = 64
NROW = 1000000

_info = plsc.get_sparse_core_info()
_NC, _NS = _info.num_cores, _info.num_subcores
_NW = _NC * _NS
_CPW = D0 // _NW
_EPW = _CPW * BATCH_SIZE
_CHUNK = 128
_NCHUNK0 = _EPW // _CHUNK
_B1PW = BATCH_SIZE // _NW
_NCHUNK1 = _B1PW // _CHUNK


def _body(d0_hbm, d1_hbm, eidx_hbm, out0_hbm, out1_hbm,
          idx_v, rows0_v, rows1_v, sem):
    wid = lax.axis_index("s") * _NC + lax.axis_index("c")
    pltpu.sync_copy(eidx_hbm.at[pl.ds(wid * _EPW, _EPW)], idx_v.at[pl.ds(0, _EPW)])
    pltpu.sync_copy(eidx_hbm.at[pl.ds(D0 * BATCH_SIZE + wid * _B1PW, _B1PW)],
                    idx_v.at[pl.ds(_EPW, _B1PW)])
    copies = []
    for j in range(_NCHUNK0):
        sl = pl.ds(j * _CHUNK, _CHUNK)
        copies.append(pltpu.async_copy(d0_hbm.at[idx_v.at[sl]], rows0_v.at[sl], sem))
    for j in range(_NCHUNK1):
        sl = pl.ds(_EPW + j * _CHUNK, _CHUNK)
        slo = pl.ds(j * _CHUNK, _CHUNK)
        copies.append(pltpu.async_copy(d1_hbm.at[idx_v.at[sl]], rows1_v.at[slo], sem))
    for c in copies:
        c.wait()
    pltpu.sync_copy(rows0_v, out0_hbm.at[pl.ds(wid * _EPW, _EPW)])
    pltpu.sync_copy(rows1_v, out1_hbm.at[pl.ds(wid * _B1PW, _B1PW)])


@jax.jit
def _run(d0flat, d1flat, eidx):
    mesh = plsc.VectorSubcoreMesh(core_axis_name="c", subcore_axis_name="s")
    f = functools.partial(
        pl.kernel,
        mesh=mesh,
        out_type=(
            jax.ShapeDtypeStruct((D0 * BATCH_SIZE,), jnp.float32),
            jax.ShapeDtypeStruct((BATCH_SIZE,), jnp.float32),
        ),
        scratch_types=[
            pltpu.VMEM((_EPW + _B1PW,), jnp.int32),
            pltpu.VMEM((_EPW,), jnp.float32),
            pltpu.VMEM((_B1PW,), jnp.float32),
            pltpu.SemaphoreType.DMA,
        ],
        compiler_params=pltpu.CompilerParams(use_tc_tiling_on_sc=False),
    )(_body)
    return f(d0flat, d1flat, eidx)


def kernel(data0, data1, step):
    loader_key = jax.random.key(42)
    key = jax.random.fold_in(loader_key, step)
    idx = jax.random.randint(key, (BATCH_SIZE,), minval=0,
                             maxval=data0.shape[0], dtype=jnp.int32)
    offs = jnp.concatenate([jnp.arange(D0, dtype=jnp.int32) * NROW,
                            jnp.zeros((1,), jnp.int32)])
    eidx = (offs[:, None] + idx[None, :]).reshape(-1)
    out0, out1 = _run(data0.T.reshape(-1), data1.reshape(-1), eidx)
    return out0.reshape(D0, BATCH_SIZE).T, out1.reshape(BATCH_SIZE, 1)


# speedup vs baseline: 1.0008x; 1.0008x over previous
import functools

import jax
import jax.numpy as jnp
from jax import lax
from jax.experimental import pallas as pl
from jax.experimental.pallas import tpu as pltpu
from jax.experimental.pallas import tpu_sc as plsc

BATCH_SIZE = 16384
D0 = 64

_info = plsc.get_sparse_core_info()
_NC, _NS = _info.num_cores, _info.num_subcores
_NW = _NC * _NS
_CPW = D0 // _NW
_CHUNK = 128
_NCHUNK0 = BATCH_SIZE // _CHUNK
_B1PW = BATCH_SIZE // _NW
_NCHUNK1 = _B1PW // _CHUNK


def _body(d0_hbm, d1_hbm, idx_hbm, out0_hbm, out1_hbm,
          idx_v, rows0_v, rows1_v, sem):
    wid = lax.axis_index("s") * _NC + lax.axis_index("c")
    c0 = wid * _CPW
    pltpu.sync_copy(idx_hbm, idx_v)
    copies = []
    for cc in range(_CPW):
        for j in range(_NCHUNK0):
            sl = pl.ds(j * _CHUNK, _CHUNK)
            copies.append(pltpu.async_copy(
                d0_hbm.at[c0 + cc].at[idx_v.at[sl]],
                rows0_v.at[cc].at[sl], sem))
    for j in range(_NCHUNK1):
        sl = pl.ds(wid * _B1PW + j * _CHUNK, _CHUNK)
        slo = pl.ds(j * _CHUNK, _CHUNK)
        copies.append(pltpu.async_copy(
            d1_hbm.at[idx_v.at[sl]], rows1_v.at[slo], sem))
    for c in copies:
        c.wait()
    pltpu.sync_copy(rows0_v, out0_hbm.at[pl.ds(c0, _CPW)])
    pltpu.sync_copy(rows1_v, out1_hbm.at[pl.ds(wid * _B1PW, _B1PW)])


@jax.jit
def _run(d0t, d1flat, idx):
    mesh = plsc.VectorSubcoreMesh(core_axis_name="c", subcore_axis_name="s")
    f = functools.partial(
        pl.kernel,
        mesh=mesh,
        out_type=(
            jax.ShapeDtypeStruct((D0, BATCH_SIZE), jnp.float32),
            jax.ShapeDtypeStruct((BATCH_SIZE,), jnp.float32),
        ),
        scratch_types=[
            pltpu.VMEM((BATCH_SIZE,), jnp.int32),
            pltpu.VMEM((_CPW, BATCH_SIZE), jnp.float32),
            pltpu.VMEM((_B1PW,), jnp.float32),
            pltpu.SemaphoreType.DMA,
        ],
        compiler_params=pltpu.CompilerParams(use_tc_tiling_on_sc=False),
    )(_body)
    return f(d0t, d1flat, idx)


def kernel(data0, data1, step):
    loader_key = jax.random.key(42)
    key = jax.random.fold_in(loader_key, step)
    idx = jax.random.randint(key, (BATCH_SIZE,), minval=0,
                             maxval=data0.shape[0], dtype=jnp.int32)
    out0, out1 = _run(data0.T, data1.reshape(-1), idx)
    return out0.T, out1.reshape(BATCH_SIZE, 1)


# speedup vs baseline: 7.8649x; 7.8583x over previous
import functools

import jax
import jax.numpy as jnp
from jax import lax
from jax.experimental import pallas as pl
from jax.experimental.pallas import tpu as pltpu
from jax.experimental.pallas import tpu_sc as plsc

BATCH_SIZE = 16384
D0 = 64

_info = plsc.get_sparse_core_info()
_NC, _NS = _info.num_cores, _info.num_subcores
_NW = _NC * _NS
_BPW = BATCH_SIZE // _NW
_CHUNK = 128
_NCHUNK = _BPW // _CHUNK
_L = 16


def _body0(d2_hbm, i2_hbm, h64_hbm, out_hbm, i2_v, h64_v, stage_v, out_v, sem):
    wid = lax.axis_index("s") * _NC + lax.axis_index("c")
    pltpu.sync_copy(i2_hbm.at[wid], i2_v)
    pltpu.sync_copy(h64_hbm.at[wid], h64_v)
    copies = []
    for j in range(_NCHUNK):
        sl = pl.ds(j * _CHUNK, _CHUNK)
        copies.append(pltpu.async_copy(
            d2_hbm.at[i2_v.at[j]], stage_v.at[sl], sem))
    for c in copies:
        c.wait()
    for w in range(_BPW // _L):
        ivec = jax.lax.iota(jnp.int32, _L) + w * _L
        hvec = h64_v[pl.ds(w * _L, _L)]
        for c in range(D0):
            cvec = jax.lax.iota(jnp.int32, _L) * 0 + c
            vals = plsc.load_gather(stage_v, [ivec, hvec + c])
            plsc.store_scatter(out_v, [cvec, ivec], vals)
    pltpu.sync_copy(out_v, out_hbm.at[:, pl.ds(wid * _BPW, _BPW)])


def _body1(d1_hbm, idx_hbm, out_hbm, idx_v, rows_v, sem):
    wid = lax.axis_index("s") * _NC + lax.axis_index("c")
    pltpu.sync_copy(idx_hbm.at[wid], idx_v)
    copies = []
    for j in range(_NCHUNK):
        sl = pl.ds(j * _CHUNK, _CHUNK)
        copies.append(pltpu.async_copy(
            d1_hbm.at[idx_v.at[j]], rows_v.at[sl], sem))
    for c in copies:
        c.wait()
    pltpu.sync_copy(rows_v, out_hbm.at[pl.ds(wid * _BPW, _BPW)])


@jax.jit
def _run(d2, d1flat, i2, h64, idx3):
    mesh = plsc.VectorSubcoreMesh(core_axis_name="c", subcore_axis_name="s")
    f0 = functools.partial(
        pl.kernel,
        mesh=mesh,
        out_type=jax.ShapeDtypeStruct((D0, BATCH_SIZE), jnp.float32),
        scratch_types=[
            pltpu.VMEM((_NCHUNK, _CHUNK), jnp.int32),
            pltpu.VMEM((_BPW,), jnp.int32),
            pltpu.VMEM((_BPW, 128), jnp.float32),
            pltpu.VMEM((D0, _BPW), jnp.float32),
            pltpu.SemaphoreType.DMA,
        ],
        compiler_params=pltpu.CompilerParams(needs_layout_passes=False),
    )(_body0)
    f1 = functools.partial(
        pl.kernel,
        mesh=mesh,
        out_type=jax.ShapeDtypeStruct((BATCH_SIZE,), jnp.float32),
        scratch_types=[
            pltpu.VMEM((_NCHUNK, _CHUNK), jnp.int32),
            pltpu.VMEM((_BPW,), jnp.float32),
            pltpu.SemaphoreType.DMA,
        ],
        compiler_params=pltpu.CompilerParams(use_tc_tiling_on_sc=False),
    )(_body1)
    out0 = f0(d2, i2, h64)
    out1 = f1(d1flat, idx3)
    return out0, out1


def kernel(data0, data1, step):
    loader_key = jax.random.key(42)
    key = jax.random.fold_in(loader_key, step)
    idx = jax.random.randint(key, (BATCH_SIZE,), minval=0,
                             maxval=data0.shape[0], dtype=jnp.int32)
    i2 = (idx >> 1).reshape(_NW, _NCHUNK, _CHUNK)
    h64 = ((idx & 1) * D0).reshape(_NW, _BPW)
    idx3 = idx.reshape(_NW, _NCHUNK, _CHUNK)
    out0, out1 = _run(data0.reshape(500000, 128), data1.reshape(-1),
                      i2, h64, idx3)
    return out0.T, out1.reshape(BATCH_SIZE, 1)


# speedup vs baseline: 11.0394x; 1.4036x over previous
import functools

import jax
import jax.numpy as jnp
from jax import lax
from jax.experimental import pallas as pl
from jax.experimental.pallas import tpu as pltpu
from jax.experimental.pallas import tpu_sc as plsc

BATCH_SIZE = 16384
D0 = 64

_info = plsc.get_sparse_core_info()
_NC, _NS = _info.num_cores, _info.num_subcores
_NW = _NC * _NS
_BPW = BATCH_SIZE // _NW
_L = 16
_PHASE = 64
_NPHASE = _BPW // _PHASE
_CHUNK = 128
_NCHUNK = _BPW // _CHUNK


def _body0(d_hbm, idx_hbm, out_hbm, idx_v, ring_v, out_v, sem):
    wid = lax.axis_index("s") * _NC + lax.axis_index("c")
    pltpu.sync_copy(idx_hbm.at[wid], idx_v)

    def extract_phase(p):
        def win(w):
            base = p * _PHASE + w * _L
            ivec = jax.lax.iota(jnp.int32, _L) + base
            svec = idx_v[pl.ds(base, _L)] & 7
            slotvec = jax.lax.iota(jnp.int32, _L) + w * _L
            for c in range(D0):
                cvec = jax.lax.iota(jnp.int32, _L) * 0 + c
                vals = plsc.load_gather(ring_v, [slotvec, svec, cvec])
                plsc.store_scatter(out_v, [cvec, ivec], vals)
        pl.loop(0, _PHASE // _L)(win)

    for p in range(_NPHASE):
        descs = []
        for w in range(_PHASE // _L):
            vec = idx_v[pl.ds(p * _PHASE + w * _L, _L)]
            for t in range(_L):
                r = vec[t]
                t8 = pl.multiple_of((r >> 3) * 8, 8)
                descs.append(pltpu.async_copy(
                    d_hbm.at[pl.ds(t8, 8), :], ring_v.at[w * _L + t], sem))
        for d in descs:
            d.wait()
        extract_phase(p)

    pltpu.sync_copy(out_v, out_hbm.at[:, pl.ds(wid * _BPW, _BPW)])


def _body1(d1_hbm, idx_hbm, out_hbm, idx_v, rows_v, sem):
    wid = lax.axis_index("s") * _NC + lax.axis_index("c")
    pltpu.sync_copy(idx_hbm.at[wid], idx_v)
    copies = []
    for j in range(_NCHUNK):
        sl = pl.ds(j * _CHUNK, _CHUNK)
        copies.append(pltpu.async_copy(
            d1_hbm.at[idx_v.at[j]], rows_v.at[sl], sem))
    for c in copies:
        c.wait()
    pltpu.sync_copy(rows_v, out_hbm.at[pl.ds(wid * _BPW, _BPW)])


@jax.jit
def _run(data0, d1flat, idx2, idx3):
    mesh = plsc.VectorSubcoreMesh(core_axis_name="c", subcore_axis_name="s")
    f0 = functools.partial(
        pl.kernel,
        mesh=mesh,
        out_type=jax.ShapeDtypeStruct((D0, BATCH_SIZE), jnp.float32),
        scratch_types=[
            pltpu.VMEM((_BPW,), jnp.int32),
            pltpu.VMEM((_PHASE, 8, D0), jnp.float32),
            pltpu.VMEM((D0, _BPW), jnp.float32),
            pltpu.SemaphoreType.DMA,
        ],
        compiler_params=pltpu.CompilerParams(needs_layout_passes=False),
    )(_body0)
    f1 = functools.partial(
        pl.kernel,
        mesh=mesh,
        out_type=jax.ShapeDtypeStruct((BATCH_SIZE,), jnp.float32),
        scratch_types=[
            pltpu.VMEM((_NCHUNK, _CHUNK), jnp.int32),
            pltpu.VMEM((_BPW,), jnp.float32),
            pltpu.SemaphoreType.DMA,
        ],
        compiler_params=pltpu.CompilerParams(use_tc_tiling_on_sc=False),
    )(_body1)
    out0 = f0(data0, idx2)
    out1 = f1(d1flat, idx3)
    return out0, out1


def kernel(data0, data1, step):
    loader_key = jax.random.key(42)
    key = jax.random.fold_in(loader_key, step)
    idx = jax.random.randint(key, (BATCH_SIZE,), minval=0,
                             maxval=data0.shape[0], dtype=jnp.int32)
    idx2 = idx.reshape(_NW, _BPW)
    idx3 = idx.reshape(_NW, _NCHUNK, _CHUNK)
    out0, out1 = _run(data0, data1.reshape(-1), idx2, idx3)
    return out0.T, out1.reshape(BATCH_SIZE, 1)
